# Initial kernel scaffold; baseline (speedup 1.0000x reference)
import jax
import jax.numpy as jnp
from jax.experimental import pallas as pl


def kernel(x, adj_indices, adj_values, W, b):
    raise NotImplementedError("write your pallas kernel here")


# speedup vs baseline: 1.5200x; 1.5200x over previous
import functools

import jax
import jax.numpy as jnp
from jax import lax
from jax.experimental import pallas as pl
from jax.experimental.pallas import tpu as pltpu
from jax.experimental.pallas import tpu_sc as plsc

_N = 10000
_T = 4
_CIN = 128
_COUT = 64
_E = 160000
_NC = 2
_NS = 16
_L = 16
_D = _COUT
_EPT = _E // _NS
_K = 80
_NCH = _EPT // _K
_RPT = _N // _NS
_ZR = 125
_NH = _N // 2
_ZPT = (_NH + 8) // _NS
_WPT = 312
_WTAIL = _NH - _NS * _WPT


def _linear_body(x_ref, w_ref, b_ref, o_ref):
    o_ref[...] = (
        jnp.dot(x_ref[...], w_ref[...], preferred_element_type=jnp.float32)
        + b_ref[...]
    )


def _linear(x2b, W2, b2):
    blk = 2000
    return pl.pallas_call(
        _linear_body,
        grid=(_NC * _N // blk,),
        in_specs=[
            pl.BlockSpec((blk, 2 * _CIN), lambda i: (i, 0)),
            pl.BlockSpec((2 * _CIN, 2 * _D), lambda i: (0, 0)),
            pl.BlockSpec((1, 2 * _D), lambda i: (0, 0)),
        ],
        out_specs=pl.BlockSpec((blk, 2 * _D), lambda i: (i, 0)),
        out_shape=jax.ShapeDtypeStruct((_NC * _N, 2 * _D), jnp.float32),
    )(x2b, W2, b2.reshape(1, 2 * _D))


def _spmm_body(y_ref, dst_ref, src_ref, val_ref, out_ref,
               grow_v, drow_v, valr_v, rows_v, half_v, zbuf_v, acc_sh, sem):
    c = lax.axis_index("c")
    s = lax.axis_index("s")
    base = s * _EPT

    def _zero_row(r, carry):
        for j in range(2 * _D // _L):
            zbuf_v[r, pl.ds(j * _L, _L)] = jnp.zeros((_L,), jnp.float32)
        return carry

    lax.fori_loop(0, _ZR, _zero_row, 0)

    for p in range(2):

        z0 = s * _ZPT
        pltpu.sync_copy(zbuf_v, acc_sh.at[pl.ds(z0, _ZR)])
        pltpu.sync_copy(zbuf_v, acc_sh.at[pl.ds(z0 + _ZR, _ZR)])
        pltpu.sync_copy(zbuf_v.at[pl.ds(0, _ZPT - 2 * _ZR)],
                        acc_sh.at[pl.ds(z0 + 2 * _ZR, _ZPT - 2 * _ZR)])

        plsc.subcore_barrier()

        lo = p * _NH

        def _chunk(ci, carry):
            off = base + ci * _K

            pltpu.sync_copy(src_ref.at[pl.ds(off, _K)], grow_v)
            pltpu.sync_copy(dst_ref.at[pl.ds(off, _K)], drow_v)
            pltpu.sync_copy(val_ref.at[pl.ds(off, _K)], valr_v)

            for j in range(_K // _L):
                grow_v[pl.ds(j * _L, _L)] = grow_v[pl.ds(j * _L, _L)] * 2 + c
                d16 = drow_v[pl.ds(j * _L, _L)] - lo
                ok = (d16 >= 0) & (d16 < _NH)
                drow_v[pl.ds(j * _L, _L)] = jnp.where(ok, d16, _NH)

            pltpu.sync_copy(y_ref.at[grow_v], rows_v)

            for g in range(_K // _L):
                val16 = valr_v[pl.ds(g * _L, _L)]
                for l in range(_L):
                    v = val16[l]
                    e = g * _L + l
                    for j in range(2 * _D // _L):
                        half_v[e, pl.ds(j * _L, _L)] = (
                            rows_v[e, pl.ds(j * _L, _L)] * v
                        )

            pltpu.sync_copy(half_v, acc_sh.at[drow_v], add=True)
            return carry

        lax.fori_loop(0, _NCH, _chunk, 0)

        plsc.subcore_barrier()

        w0 = s * _WPT
        pltpu.sync_copy(acc_sh.at[pl.ds(w0, _WPT)],
                        out_ref.at[pl.ds(lo + w0, _WPT), c])

        @pl.when(s == 0)
        def _tail():
            pltpu.sync_copy(acc_sh.at[pl.ds(_NS * _WPT, _WTAIL)],
                            out_ref.at[pl.ds(lo + _NS * _WPT, _WTAIL), c])

        if p == 0:
            plsc.subcore_barrier()


_spmm = functools.partial(
    pl.kernel,
    out_type=jax.ShapeDtypeStruct((_N, _NC, 2 * _D), jnp.float32),
    mesh=plsc.VectorSubcoreMesh(
        core_axis_name="c", subcore_axis_name="s",
        num_cores=_NC, num_subcores=_NS,
    ),
    scratch_types=[
        pltpu.VMEM((_K,), jnp.int32),
        pltpu.VMEM((_K,), jnp.int32),
        pltpu.VMEM((_K,), jnp.float32),
        pltpu.VMEM((_K, 2 * _D), jnp.float32),
        pltpu.VMEM((_K, 2 * _D), jnp.float32),
        pltpu.VMEM((_ZR, 2 * _D), jnp.float32),
        pltpu.VMEM_SHARED((_NH + 8, 2 * _D), jnp.float32),
        pltpu.SemaphoreType.DMA,
    ],
)(_spmm_body)


@jax.jit
def kernel(x, adj_indices, adj_values, W, b):
    x2b = x.reshape(_NC * _N, 2 * _CIN)
    W2 = jnp.zeros((2 * _CIN, 2 * _D), jnp.float32)
    W2 = W2.at[:_CIN, :_COUT].set(W).at[_CIN:, _COUT:].set(W)
    b2 = jnp.concatenate([b, b])
    y2 = _linear(x2b, W2, b2)
    out = _spmm(y2, adj_indices[0], adj_indices[1], adj_values)
    return out.reshape(1, _N, _T, _COUT)


# speedup vs baseline: 2.0880x; 1.3736x over previous
import functools

import jax
import jax.numpy as jnp
from jax import lax
from jax.experimental import pallas as pl
from jax.experimental.pallas import tpu as pltpu
from jax.experimental.pallas import tpu_sc as plsc

_N = 10000
_T = 4
_CIN = 128
_COUT = 64
_E = 160000
_NC = 2
_NS = 16
_L = 16
_D = _COUT
_EPT = _E // _NS
_K = 80
_NCH = _EPT // _K
_RPT = _N // _NS
_ZR = 125
_NH = _N // 2
_ZPT = (_NH + 8) // _NS
_WPT = 312
_WTAIL = _NH - _NS * _WPT


def _linear_body(x_ref, w_ref, b_ref, o_ref):
    o_ref[...] = (
        jnp.dot(x_ref[...], w_ref[...], preferred_element_type=jnp.float32)
        + b_ref[...]
    )


def _linear(x2b, W2, b2):
    blk = 2000
    return pl.pallas_call(
        _linear_body,
        grid=(_NC * _N // blk,),
        in_specs=[
            pl.BlockSpec((blk, 2 * _CIN), lambda i: (i, 0)),
            pl.BlockSpec((2 * _CIN, 2 * _D), lambda i: (0, 0)),
            pl.BlockSpec((1, 2 * _D), lambda i: (0, 0)),
        ],
        out_specs=pl.BlockSpec((blk, 2 * _D), lambda i: (i, 0)),
        out_shape=jax.ShapeDtypeStruct((_NC * _N, 2 * _D), jnp.float32),
    )(x2b, W2, b2.reshape(1, 2 * _D))


def _spmm_body(y_ref, dst_ref, src_ref, val_ref, out_ref,
               growA, drowA, valrA, rowsA, growB, drowB, valrB, rowsB,
               half_v, zbuf_v, acc_sh, semA, semB):
    c = lax.axis_index("c")
    s = lax.axis_index("s")
    base = s * _EPT

    def _zero_row(r, carry):
        for j in range(2 * _D // _L):
            zbuf_v[r, pl.ds(j * _L, _L)] = jnp.zeros((_L,), jnp.float32)
        return carry

    lax.fori_loop(0, _ZR, _zero_row, 0)

    for p in range(2):
        lo = p * _NH

        z0 = s * _ZPT
        pltpu.sync_copy(zbuf_v, acc_sh.at[pl.ds(z0, _ZR)])
        pltpu.sync_copy(zbuf_v, acc_sh.at[pl.ds(z0 + _ZR, _ZR)])
        pltpu.sync_copy(zbuf_v.at[pl.ds(0, _ZPT - 2 * _ZR)],
                        acc_sh.at[pl.ds(z0 + 2 * _ZR, _ZPT - 2 * _ZR)])

        plsc.subcore_barrier()

        def _stage_start(cidx, grow, drow, valr, rows, sm):
            off = base + cidx * _K
            pltpu.sync_copy(src_ref.at[pl.ds(off, _K)], grow)
            pltpu.sync_copy(dst_ref.at[pl.ds(off, _K)], drow)
            pltpu.sync_copy(val_ref.at[pl.ds(off, _K)], valr)
            for j in range(_K // _L):
                grow[pl.ds(j * _L, _L)] = grow[pl.ds(j * _L, _L)] * 2 + c
                d16 = drow[pl.ds(j * _L, _L)] - lo
                ok = (d16 >= 0) & (d16 < _NH)
                drow[pl.ds(j * _L, _L)] = jnp.where(ok, d16, _NH)
            pltpu.async_copy(y_ref.at[grow], rows, sm)

        def _finish(grow, drow, valr, rows, sm):
            pltpu.make_async_copy(y_ref.at[grow], rows, sm).wait()
            for g in range(_K // _L):
                val16 = valr[pl.ds(g * _L, _L)]
                for l in range(_L):
                    v = val16[l]
                    e = g * _L + l
                    for j in range(2 * _D // _L):
                        half_v[e, pl.ds(j * _L, _L)] = (
                            rows[e, pl.ds(j * _L, _L)] * v
                        )
            pltpu.sync_copy(half_v, acc_sh.at[drow], add=True)

        _stage_start(0, growA, drowA, valrA, rowsA, semA)

        def _pair(k, carry):
            _stage_start(2 * k + 1, growB, drowB, valrB, rowsB, semB)
            _finish(growA, drowA, valrA, rowsA, semA)
            _stage_start(2 * k + 2, growA, drowA, valrA, rowsA, semA)
            _finish(growB, drowB, valrB, rowsB, semB)
            return carry

        lax.fori_loop(0, (_NCH - 1) // 2, _pair, 0)
        _finish(growA, drowA, valrA, rowsA, semA)

        plsc.subcore_barrier()

        w0 = s * _WPT
        pltpu.sync_copy(acc_sh.at[pl.ds(w0, _WPT)],
                        out_ref.at[pl.ds(lo + w0, _WPT), c])

        @pl.when(s == 0)
        def _tail():
            pltpu.sync_copy(acc_sh.at[pl.ds(_NS * _WPT, _WTAIL)],
                            out_ref.at[pl.ds(lo + _NS * _WPT, _WTAIL), c])

        if p == 0:
            plsc.subcore_barrier()


_spmm = functools.partial(
    pl.kernel,
    out_type=jax.ShapeDtypeStruct((_N, _NC, 2 * _D), jnp.float32),
    mesh=plsc.VectorSubcoreMesh(
        core_axis_name="c", subcore_axis_name="s",
        num_cores=_NC, num_subcores=_NS,
    ),
    scratch_types=[
        pltpu.VMEM((_K,), jnp.int32),
        pltpu.VMEM((_K,), jnp.int32),
        pltpu.VMEM((_K,), jnp.float32),
        pltpu.VMEM((_K, 2 * _D), jnp.float32),
        pltpu.VMEM((_K,), jnp.int32),
        pltpu.VMEM((_K,), jnp.int32),
        pltpu.VMEM((_K,), jnp.float32),
        pltpu.VMEM((_K, 2 * _D), jnp.float32),
        pltpu.VMEM((_K, 2 * _D), jnp.float32),
        pltpu.VMEM((_ZR, 2 * _D), jnp.float32),
        pltpu.VMEM_SHARED((_NH + 8, 2 * _D), jnp.float32),
        pltpu.SemaphoreType.DMA,
        pltpu.SemaphoreType.DMA,
    ],
)(_spmm_body)


@jax.jit
def kernel(x, adj_indices, adj_values, W, b):
    x2b = x.reshape(_NC * _N, 2 * _CIN)
    W2 = jnp.zeros((2 * _CIN, 2 * _D), jnp.float32)
    W2 = W2.at[:_CIN, :_COUT].set(W).at[_CIN:, _COUT:].set(W)
    b2 = jnp.concatenate([b, b])
    y2 = _linear(x2b, W2, b2)
    out = _spmm(y2, adj_indices[0], adj_indices[1], adj_values)
    return out.reshape(1, _N, _T, _COUT)
